# baseline (speedup 1.0000x reference)
import functools

import jax
import jax.numpy as jnp
from jax import lax
from jax.experimental import pallas as pl
from jax.experimental.pallas import tpu as pltpu
from jax.experimental.pallas import tpu_sc as plsc

N_TOTAL = 32768
D = 1024
LANES = 16
NC, NS = 2, 16
NW = NC * NS
B_PER_W = N_TOTAL // NW
CHUNK = 16
NBUF = 4
NCHUNK = B_PER_W // CHUNK

_mesh = plsc.VectorSubcoreMesh(core_axis_name="c", subcore_axis_name="s")

_GATHER_DNUMS = lax.GatherDimensionNumbers(
    offset_dims=(), collapsed_slice_dims=(0,), start_index_map=(0,)
)


def _lane_shuffle(x, perm):
    return lax.gather(
        x, perm[:, None], _GATHER_DNUMS, (1,),
        mode=lax.GatherScatterMode.PROMISE_IN_BOUNDS,
    )


def _lane_sum(x):
    for sh in (8, 4, 2, 1):
        perm = jnp.arange(LANES, dtype=jnp.int32) ^ sh
        x = x + _lane_shuffle(x, perm)
    return x


def _rsqrt16(t):
    t = jnp.maximum(t, jnp.float32(1e-24))
    i = lax.bitcast_convert_type(t, jnp.int32)
    y = lax.bitcast_convert_type(jnp.int32(0x5F3759DF) - (i >> 1), jnp.float32)
    for _ in range(3):
        y = y * (jnp.float32(1.5) - jnp.float32(0.5) * t * y * y)
    return y


@functools.partial(
    pl.kernel,
    mesh=_mesh,
    out_type=jax.ShapeDtypeStruct((N_TOTAL, D), jnp.float32),
    scratch_types=[
        pltpu.VMEM((B_PER_W,), jnp.int32),
    ] + [pltpu.VMEM((CHUNK, D), jnp.float32)] * NBUF
      + [pltpu.SemaphoreType.DMA] * (2 * NBUF),
)
def _gather_normalize(idx_hbm, table_hbm, out_hbm, idx_v, *bufs_and_sems):
    rows = bufs_and_sems[:NBUF]
    gsem = bufs_and_sems[NBUF:2 * NBUF]
    osem = bufs_and_sems[2 * NBUF:]

    wid = lax.axis_index("s") * NC + lax.axis_index("c")
    base = wid * B_PER_W
    pltpu.sync_copy(idx_hbm.at[pl.ds(base, B_PER_W)], idx_v)

    def start_gather(j, b):
        pltpu.async_copy(
            table_hbm.at[idx_v.at[pl.ds(j * CHUNK, CHUNK)]], rows[b], gsem[b]
        )

    def start_scatter(j, b):
        pltpu.async_copy(
            rows[b], out_hbm.at[pl.ds(base + j * CHUNK, CHUNK)], osem[b]
        )

    def wait_gather(b):
        pltpu.make_async_copy(
            table_hbm.at[idx_v.at[pl.ds(0, CHUNK)]], rows[b], gsem[b]
        ).wait()

    def wait_scatter(b):
        pltpu.make_async_copy(
            rows[b], out_hbm.at[pl.ds(0, CHUNK)], osem[b]
        ).wait()

    lane_iota = jnp.arange(LANES, dtype=jnp.int32)

    def normalize_chunk(buf):
        def p1(h, z):
            for u in range(2):
                r = h * 2 + u
                accs = [jnp.zeros((LANES,), jnp.float32) for _ in range(4)]
                for j in range(D // LANES):
                    s = buf[r, pl.ds(j * LANES, LANES)]
                    accs[j % 4] = accs[j % 4] + s * s
                acc = _lane_sum((accs[0] + accs[1]) + (accs[2] + accs[3]))
                z = jnp.where(lane_iota == r, acc, z)
            return z

        z = lax.fori_loop(0, CHUNK // 2, p1, jnp.zeros((LANES,), jnp.float32))
        y = _rsqrt16(z)

        def p2(h, _):
            for u in range(2):
                r = h * 2 + u
                s = _lane_shuffle(y, jnp.full((LANES,), r, jnp.int32))
                for j in range(D // LANES):
                    sl = pl.ds(j * LANES, LANES)
                    buf[r, sl] = buf[r, sl] * s
            return 0

        lax.fori_loop(0, CHUNK // 2, p2, 0)

    start_gather(0, 0)
    start_gather(1, 1)

    def group_body(g, _):
        for b in range(NBUF):
            i = g * NBUF + b
            wait_gather(b)
            normalize_chunk(rows[b])
            start_scatter(i, b)
            j = i + 2
            bj = (b + 2) % NBUF

            @pl.when(j < NCHUNK)
            def _():
                @pl.when(j >= NBUF)
                def _():
                    wait_scatter(bj)
                start_gather(j, bj)
        return 0

    lax.fori_loop(0, NCHUNK // NBUF, group_body, 0)

    for b in range(NBUF):
        wait_scatter(b)


def kernel(positions, positional_embeddings):
    idx = positions.reshape(-1).astype(jnp.int32)
    out = _gather_normalize(idx, positional_embeddings)
    return out.reshape(positions.shape + (D,))


# speedup vs baseline: 1.4934x; 1.4934x over previous
import functools

import jax
import jax.numpy as jnp
from jax import lax
from jax.experimental import pallas as pl
from jax.experimental.pallas import tpu as pltpu
from jax.experimental.pallas import tpu_sc as plsc

N_TOTAL = 32768
D = 1024
LANES = 16
NC, NS = 2, 16
NW = NC * NS
B_PER_W = N_TOTAL // NW
CHUNK = 16
NBUF = 4
AHEAD = 2
NCHUNK = B_PER_W // CHUNK

_mesh = plsc.VectorSubcoreMesh(core_axis_name="c", subcore_axis_name="s")

_GATHER_DNUMS = lax.GatherDimensionNumbers(
    offset_dims=(), collapsed_slice_dims=(0,), start_index_map=(0,)
)


def _lane_shuffle(x, perm):
    return lax.gather(
        x, perm[:, None], _GATHER_DNUMS, (1,),
        mode=lax.GatherScatterMode.PROMISE_IN_BOUNDS,
    )


def _lane_sum(x):
    for sh in (8, 4, 2, 1):
        perm = jnp.arange(LANES, dtype=jnp.int32) ^ sh
        x = x + _lane_shuffle(x, perm)
    return x


def _rsqrt16(t):
    t = jnp.maximum(t, jnp.float32(1e-24))
    i = lax.bitcast_convert_type(t, jnp.int32)
    y = lax.bitcast_convert_type(jnp.int32(0x5F3759DF) - (i >> 1), jnp.float32)
    for _ in range(3):
        y = y * (jnp.float32(1.5) - jnp.float32(0.5) * t * y * y)
    return y


@functools.partial(
    pl.kernel,
    mesh=_mesh,
    out_type=jax.ShapeDtypeStruct((N_TOTAL, D), jnp.float32),
    scratch_types=[
        pltpu.VMEM((B_PER_W,), jnp.int32),
        pltpu.VMEM((NBUF, CHUNK, D), jnp.float32),
        pltpu.SemaphoreType.DMA((NBUF,)),
        pltpu.SemaphoreType.DMA((NBUF,)),
    ],
)
def _gather_normalize(idx_hbm, table_hbm, out_hbm, idx_v, rows, gsem, osem):
    wid = lax.axis_index("s") * NC + lax.axis_index("c")
    base = wid * B_PER_W
    pltpu.sync_copy(idx_hbm.at[pl.ds(base, B_PER_W)], idx_v)

    def start_gather(j, b):
        pltpu.async_copy(
            table_hbm.at[idx_v.at[pl.ds(j * CHUNK, CHUNK)]],
            rows.at[b], gsem.at[b],
        )

    def start_scatter(j, b):
        pltpu.async_copy(
            rows.at[b], out_hbm.at[pl.ds(base + j * CHUNK, CHUNK)], osem.at[b]
        )

    def wait_gather(b):
        pltpu.make_async_copy(
            table_hbm.at[idx_v.at[pl.ds(0, CHUNK)]], rows.at[b], gsem.at[b]
        ).wait()

    def wait_scatter(b):
        pltpu.make_async_copy(
            rows.at[b], out_hbm.at[pl.ds(0, CHUNK)], osem.at[b]
        ).wait()

    lane_iota = jnp.arange(LANES, dtype=jnp.int32)

    def normalize_chunk(b):
        def p1(r, z):
            accs = [jnp.zeros((LANES,), jnp.float32) for _ in range(4)]
            for j in range(D // LANES):
                s = rows[b, r, pl.ds(j * LANES, LANES)]
                accs[j % 4] = accs[j % 4] + s * s
            acc = _lane_sum((accs[0] + accs[1]) + (accs[2] + accs[3]))
            return jnp.where(lane_iota == r, acc, z)

        z = lax.fori_loop(0, CHUNK, p1, jnp.zeros((LANES,), jnp.float32))
        y = _rsqrt16(z)

        def p2(r, _):
            s = _lane_shuffle(y, jnp.full((LANES,), r, jnp.int32))
            for j in range(D // LANES):
                sl = pl.ds(j * LANES, LANES)
                rows[b, r, sl] = rows[b, r, sl] * s
            return 0

        lax.fori_loop(0, CHUNK, p2, 0)

    for j in range(AHEAD):
        start_gather(j, j)

    def chunk_body(i, _):
        b = lax.rem(i, NBUF)
        wait_gather(b)
        normalize_chunk(b)
        start_scatter(i, b)
        j = i + AHEAD
        bj = lax.rem(j, NBUF)

        @pl.when(j < NCHUNK)
        def _():
            @pl.when(j >= NBUF)
            def _():
                wait_scatter(bj)
            start_gather(j, bj)
        return 0

    lax.fori_loop(0, NCHUNK, chunk_body, 0)

    for b in range(NBUF):
        wait_scatter(b)


def kernel(positions, positional_embeddings):
    idx = positions.reshape(-1).astype(jnp.int32)
    out = _gather_normalize(idx, positional_embeddings)
    return out.reshape(positions.shape + (D,))


# speedup vs baseline: 1.6182x; 1.0836x over previous
import functools

import jax
import jax.numpy as jnp
from jax import lax
from jax.experimental import pallas as pl
from jax.experimental.pallas import tpu as pltpu
from jax.experimental.pallas import tpu_sc as plsc

N_TOTAL = 32768
D = 1024
LANES = 16
NC, NS = 2, 16
NW = NC * NS
B_PER_W = N_TOTAL // NW
CHUNK = 16
NBUF = 6
AHEAD = 3
NCHUNK = B_PER_W // CHUNK

_mesh = plsc.VectorSubcoreMesh(core_axis_name="c", subcore_axis_name="s")

_GATHER_DNUMS = lax.GatherDimensionNumbers(
    offset_dims=(), collapsed_slice_dims=(0,), start_index_map=(0,)
)


def _lane_shuffle(x, perm):
    return lax.gather(
        x, perm[:, None], _GATHER_DNUMS, (1,),
        mode=lax.GatherScatterMode.PROMISE_IN_BOUNDS,
    )


def _lane_sum(x):
    for sh in (8, 4, 2, 1):
        perm = jnp.arange(LANES, dtype=jnp.int32) ^ sh
        x = x + _lane_shuffle(x, perm)
    return x


def _rsqrt16(t):
    t = jnp.maximum(t, jnp.float32(1e-24))
    i = lax.bitcast_convert_type(t, jnp.int32)
    y = lax.bitcast_convert_type(jnp.int32(0x5F3759DF) - (i >> 1), jnp.float32)
    for _ in range(3):
        y = y * (jnp.float32(1.5) - jnp.float32(0.5) * t * y * y)
    return y


@functools.partial(
    pl.kernel,
    mesh=_mesh,
    out_type=jax.ShapeDtypeStruct((N_TOTAL, D), jnp.float32),
    scratch_types=[
        pltpu.VMEM((B_PER_W,), jnp.int32),
        pltpu.VMEM((NBUF, CHUNK, D), jnp.float32),
        pltpu.SemaphoreType.DMA((NBUF,)),
        pltpu.SemaphoreType.DMA((NBUF,)),
    ],
)
def _gather_normalize(idx_hbm, table_hbm, out_hbm, idx_v, rows, gsem, osem):
    wid = lax.axis_index("s") * NC + lax.axis_index("c")
    base = wid * B_PER_W
    pltpu.sync_copy(idx_hbm.at[pl.ds(base, B_PER_W)], idx_v)

    def start_gather(j, b):
        pltpu.async_copy(
            table_hbm.at[idx_v.at[pl.ds(j * CHUNK, CHUNK)]],
            rows.at[b], gsem.at[b],
        )

    def start_scatter(j, b):
        pltpu.async_copy(
            rows.at[b], out_hbm.at[pl.ds(base + j * CHUNK, CHUNK)], osem.at[b]
        )

    def wait_gather(b):
        pltpu.make_async_copy(
            table_hbm.at[idx_v.at[pl.ds(0, CHUNK)]], rows.at[b], gsem.at[b]
        ).wait()

    def wait_scatter(b):
        pltpu.make_async_copy(
            rows.at[b], out_hbm.at[pl.ds(0, CHUNK)], osem.at[b]
        ).wait()

    lane_iota = jnp.arange(LANES, dtype=jnp.int32)

    def normalize_chunk(b):
        def p1(r, z):
            accs = [jnp.zeros((LANES,), jnp.float32) for _ in range(4)]
            for j in range(D // LANES):
                s = rows[b, r, pl.ds(j * LANES, LANES)]
                accs[j % 4] = accs[j % 4] + s * s
            acc = _lane_sum((accs[0] + accs[1]) + (accs[2] + accs[3]))
            return jnp.where(lane_iota == r, acc, z)

        z = lax.fori_loop(0, CHUNK, p1, jnp.zeros((LANES,), jnp.float32))
        y = _rsqrt16(z)

        def p2(r, _):
            s = _lane_shuffle(y, jnp.full((LANES,), r, jnp.int32))
            for j in range(D // LANES):
                sl = pl.ds(j * LANES, LANES)
                rows[b, r, sl] = rows[b, r, sl] * s
            return 0

        lax.fori_loop(0, CHUNK, p2, 0)

    for j in range(AHEAD):
        start_gather(j, j)

    def chunk_body(i, _):
        b = lax.rem(i, NBUF)
        wait_gather(b)
        normalize_chunk(b)
        start_scatter(i, b)
        j = i + AHEAD
        bj = lax.rem(j, NBUF)

        @pl.when(j < NCHUNK)
        def _():
            @pl.when(j >= NBUF)
            def _():
                wait_scatter(bj)
            start_gather(j, bj)
        return 0

    lax.fori_loop(0, NCHUNK, chunk_body, 0)

    for b in range(NBUF):
        wait_scatter(b)


def kernel(positions, positional_embeddings):
    idx = positions.reshape(-1).astype(jnp.int32)
    out = _gather_normalize(idx, positional_embeddings)
    return out.reshape(positions.shape + (D,))
